# baseline (speedup 1.0000x reference)
import functools

import jax
import jax.numpy as jnp
from jax import lax
from jax.experimental import pallas as pl
from jax.experimental.pallas import tpu as pltpu
from jax.experimental.pallas import tpu_sc as plsc

N_SPEAKERS = 10000
EMBED = 512
HIDDEN = 1024
BATCH = 16384


_BM = 2048


def _mm_body(a_ref, w_ref, b_ref, o_ref):
    o_ref[...] = (
        lax.dot_general(
            a_ref[...], w_ref[...],
            (((1,), (1,)), ((), ())),
            preferred_element_type=jnp.float32,
        )
        + b_ref[...]
    )


def _project(rows, w, b2d):
    n = rows.shape[0]
    return pl.pallas_call(
        _mm_body,
        grid=(n // _BM,),
        in_specs=[
            pl.BlockSpec((_BM, EMBED), lambda i: (i, 0)),
            pl.BlockSpec((HIDDEN, EMBED), lambda i: (0, 0)),
            pl.BlockSpec((1, HIDDEN), lambda i: (0, 0)),
        ],
        out_specs=pl.BlockSpec((_BM, HIDDEN), lambda i: (i, 0)),
        out_shape=jax.ShapeDtypeStruct((n, HIDDEN), jnp.float32),
    )(rows, w, b2d)


_NC = 2
_NS = 16
_NW = _NC * _NS
_B_PER_W = BATCH // _NW
_C = 32
_NCH = _B_PER_W // _C
_NBUF = 3

_EW = EMBED // 2

_sc_mesh = plsc.VectorSubcoreMesh(core_axis_name="c", subcore_axis_name="s")


@functools.partial(
    pl.kernel,
    mesh=_sc_mesh,
    compiler_params=pltpu.CompilerParams(use_tc_tiling_on_sc=False),
    out_type=jax.ShapeDtypeStruct((BATCH, _EW), jnp.int32),
    scratch_types=[
        pltpu.VMEM((_B_PER_W,), jnp.int32),
        pltpu.VMEM((_C, _EW), jnp.int32),
        pltpu.VMEM((_C, _EW), jnp.int32),
        pltpu.VMEM((_C, _EW), jnp.int32),
        pltpu.SemaphoreType.DMA,
        pltpu.SemaphoreType.DMA,
        pltpu.SemaphoreType.DMA,
        pltpu.SemaphoreType.DMA,
        pltpu.SemaphoreType.DMA,
        pltpu.SemaphoreType.DMA,
    ],
)
def _sc_gather(ids_hbm, tab_hbm, out_hbm, idx_v,
               buf0, buf1, buf2, sg0, sg1, sg2, ss0, ss1, ss2):
    wid = lax.axis_index("s") * _NC + lax.axis_index("c")
    base = wid * _B_PER_W
    pltpu.sync_copy(ids_hbm.at[pl.ds(base, _B_PER_W)], idx_v)

    bufs = (buf0, buf1, buf2)
    sg = (sg0, sg1, sg2)
    ss = (ss0, ss1, ss2)

    def start_gather(c):
        return pltpu.async_copy(
            tab_hbm.at[idx_v.at[pl.ds(c * _C, _C)]], bufs[c % _NBUF], sg[c % _NBUF]
        )

    def start_scatter(c):
        return pltpu.async_copy(
            bufs[c % _NBUF], out_hbm.at[pl.ds(base + c * _C, _C)], ss[c % _NBUF]
        )

    gathers = [None] * _NCH
    scatters = [None] * _NCH
    gathers[0] = start_gather(0)
    gathers[1] = start_gather(1)
    for c in range(_NCH):
        gathers[c].wait()
        scatters[c] = start_scatter(c)
        nxt = c + 2
        if nxt < _NCH:
            if c >= 1:
                scatters[c - 1].wait()
            gathers[nxt] = start_gather(nxt)
    for c in range(_NCH - 3, _NCH):
        scatters[c].wait()


def kernel(speaker_id, speaker_table, proj_w, proj_b):
    ids = speaker_id.astype(jnp.int32)
    tab16 = speaker_table.astype(jnp.bfloat16)
    tab_i32 = lax.bitcast_convert_type(
        tab16.reshape(N_SPEAKERS, _EW, 2), jnp.int32)
    w16 = proj_w.astype(jnp.bfloat16)
    emb_i32 = _sc_gather(ids, tab_i32)
    emb = lax.bitcast_convert_type(emb_i32, jnp.bfloat16).reshape(BATCH, EMBED)
    return _project(emb, w16, proj_b.reshape(1, HIDDEN))


# speedup vs baseline: 5.4357x; 5.4357x over previous
import functools

import jax
import jax.numpy as jnp
from jax import lax
from jax.experimental import pallas as pl
from jax.experimental.pallas import tpu as pltpu
from jax.experimental.pallas import tpu_sc as plsc

N_SPEAKERS = 10000
EMBED = 512
HIDDEN = 1024
BATCH = 16384


_BM = 4096


def _mm_body(a_ref, w_ref, b_ref, o_ref):
    o_ref[...] = (
        lax.dot_general(
            a_ref[...], w_ref[...],
            (((1,), (1,)), ((), ())),
            preferred_element_type=jnp.float32,
        )
        + b_ref[...]
    )


def _project(rows, w, b2d):
    n = rows.shape[0]
    return pl.pallas_call(
        _mm_body,
        grid=(n // _BM,),
        in_specs=[
            pl.BlockSpec((_BM, EMBED), lambda i: (i, 0)),
            pl.BlockSpec((HIDDEN, EMBED), lambda i: (0, 0)),
            pl.BlockSpec((1, HIDDEN), lambda i: (0, 0)),
        ],
        out_specs=pl.BlockSpec((_BM, HIDDEN), lambda i: (i, 0)),
        out_shape=jax.ShapeDtypeStruct((n, HIDDEN), jnp.float32),
    )(rows, w, b2d)


_NC = 2
_NS = 16
_NW = _NC * _NS
_B_PER_W = BATCH // _NW
_C = 64
_NCH = _B_PER_W // _C
_NBUF = 3

_sc_mesh = plsc.VectorSubcoreMesh(core_axis_name="c", subcore_axis_name="s")


@functools.partial(
    pl.kernel,
    mesh=_sc_mesh,
    out_type=jax.ShapeDtypeStruct((BATCH, EMBED), jnp.float32),
    scratch_types=[
        pltpu.VMEM((_B_PER_W,), jnp.int32),
        pltpu.VMEM((_C, EMBED), jnp.float32),
        pltpu.VMEM((_C, EMBED), jnp.float32),
        pltpu.VMEM((_C, EMBED), jnp.float32),
        pltpu.SemaphoreType.DMA,
        pltpu.SemaphoreType.DMA,
        pltpu.SemaphoreType.DMA,
        pltpu.SemaphoreType.DMA,
        pltpu.SemaphoreType.DMA,
        pltpu.SemaphoreType.DMA,
    ],
)
def _sc_gather(ids_hbm, tab_hbm, out_hbm, idx_v,
               buf0, buf1, buf2, sg0, sg1, sg2, ss0, ss1, ss2):
    wid = lax.axis_index("s") * _NC + lax.axis_index("c")
    base = wid * _B_PER_W
    pltpu.sync_copy(ids_hbm.at[pl.ds(base, _B_PER_W)], idx_v)

    bufs = (buf0, buf1, buf2)
    sg = (sg0, sg1, sg2)
    ss = (ss0, ss1, ss2)

    def start_gather(c):
        return pltpu.async_copy(
            tab_hbm.at[idx_v.at[pl.ds(c * _C, _C)]], bufs[c % _NBUF], sg[c % _NBUF]
        )

    def start_scatter(c):
        return pltpu.async_copy(
            bufs[c % _NBUF], out_hbm.at[pl.ds(base + c * _C, _C)], ss[c % _NBUF]
        )

    gathers = [None] * _NCH
    scatters = [None] * _NCH
    gathers[0] = start_gather(0)
    gathers[1] = start_gather(1)
    for c in range(_NCH):
        gathers[c].wait()
        scatters[c] = start_scatter(c)
        nxt = c + 2
        if nxt < _NCH:
            if c >= 1:
                scatters[c - 1].wait()
            gathers[nxt] = start_gather(nxt)
    for c in range(_NCH - 3, _NCH):
        scatters[c].wait()


def kernel(speaker_id, speaker_table, proj_w, proj_b):
    ids = speaker_id.astype(jnp.int32)
    emb = _sc_gather(ids, speaker_table)
    return _project(emb, proj_w, proj_b.reshape(1, HIDDEN))
